# baseline (speedup 1.0000x reference)
import jax
import jax.numpy as jnp
from jax.experimental import pallas as pl

_B, _N, _DIM = 32, 8192, 128
_K = 32
_ROWS = _N // 128
_BC = 4
_HCH = 4096


def _fast_exp(t):
    u = jnp.maximum(t, -30.0) * (1.0 / 128.0)
    p = 1.0 + u * (1.0 + u * (0.5 + u * (1.0 / 6.0 + u * (
        1.0 / 24.0 + u * (1.0 / 120.0 + u * (1.0 / 720.0))))))
    for _ in range(7):
        p = p * p
    return p


def _wta_kernel(x_ref, w_ref, b_ref, out_ref, mask_ref):
    nh = _N // _HCH
    bbar = jnp.mean(b_ref[...])
    s_list = []
    for cb in range(_BC):
        s_parts = []
        for h in range(nh):
            proj = jax.lax.dot_general(
                x_ref[cb, h * _HCH:(h + 1) * _HCH, :], w_ref[...],
                dimension_numbers=(((1,), (1,)), ((), ())),
                preferred_element_type=jnp.float32,
            )
            proj3 = proj.reshape(_HCH // 128, 128, _DIM)
            s_parts.append(jnp.mean(proj3, axis=-1) + bbar)
        s_list.append(jnp.concatenate(s_parts, axis=0))
    s4 = jnp.stack(s_list)

    m0 = jnp.max(s4, axis=(1, 2), keepdims=True)
    e4 = _fast_exp(s4 - m0)
    z4 = jnp.sum(e4, axis=(1, 2), keepdims=True)

    lin = (jax.lax.broadcasted_iota(jnp.int32, (1, _ROWS, 128), 1) * 128
           + jax.lax.broadcasted_iota(jnp.int32, (1, _ROWS, 128), 2))
    big = jnp.int32(2 ** 30)
    neg = jnp.float32(-jnp.inf)

    def body(_, carry):
        sw, msk = carry
        m = jnp.max(sw, axis=(1, 2), keepdims=True)
        idx = jnp.min(jnp.where(sw == m, lin, big), axis=(1, 2), keepdims=True)
        chosen = lin == idx
        msk = jnp.where(chosen, 1.0, msk)
        sw = jnp.where(chosen, neg, sw)
        return sw, msk

    _, msk4 = jax.lax.fori_loop(
        0, _K, body, (s4, jnp.zeros((_BC, _ROWS, 128), jnp.float32)))

    w4 = e4 * msk4
    rh = _HCH // 128
    for cb in range(_BC):
        acc = jnp.zeros((_DIM,), jnp.float32)
        for h in range(nh):
            x3 = x_ref[cb, h * _HCH:(h + 1) * _HCH, :].reshape(rh, 128, _DIM)
            wh = w4[cb, h * rh:(h + 1) * rh]
            prod = (x3 * wh[:, :, None]).reshape(_HCH, _DIM)
            acc = acc + jnp.sum(prod, axis=0)
        out_ref[cb, 0, :] = acc * (1.0 / z4[cb, 0, 0])
    mask_ref[...] = msk4


def kernel(x, W, b):
    out, mask3 = pl.pallas_call(
        _wta_kernel,
        grid=(_B // _BC,),
        in_specs=[
            pl.BlockSpec((_BC, _N, _DIM), lambda i: (i, 0, 0)),
            pl.BlockSpec((_DIM, _DIM), lambda i: (0, 0)),
            pl.BlockSpec((_DIM,), lambda i: (0,)),
        ],
        out_specs=[
            pl.BlockSpec((_BC, 1, _DIM), lambda i: (i, 0, 0)),
            pl.BlockSpec((_BC, _ROWS, 128), lambda i: (i, 0, 0)),
        ],
        out_shape=[
            jax.ShapeDtypeStruct((_B, 1, _DIM), jnp.float32),
            jax.ShapeDtypeStruct((_B, _ROWS, 128), jnp.float32),
        ],
    )(x, W, b)
    return out.reshape(_B, _DIM), mask3.reshape(_B, _N)
